# baseline (speedup 1.0000x reference)
import functools

import jax
import jax.numpy as jnp
from jax import lax
from jax.experimental import pallas as pl
from jax.experimental.pallas import tpu as pltpu
from jax.experimental.pallas import tpu_sc as plsc

VOCAB = 1000
DIM = 64
SENT = 50
B = 4096
OUT = SENT + 1
WPR = DIM // 2
VCHUNK = 64

try:
    _info = plsc.get_sparse_core_info()
    _NC, _NS, _L = _info.num_cores, _info.num_subcores, _info.num_lanes
except Exception:
    _NC, _NS, _L = 2, 16, 16

NW = _NC * _NS
BPW = B // NW
NBG = BPW // _L

_mesh = plsc.VectorSubcoreMesh(
    core_axis_name="c", subcore_axis_name="s",
    num_cores=_NC, num_subcores=_NS,
)

_TGROUPS = [(0, range(_L)), (_L, range(_L)), (2 * _L, range(_L)),
            (3 * _L, range(SENT - 3 * _L))]


@functools.partial(
    pl.kernel,
    out_type=jax.ShapeDtypeStruct((B, DIM), jnp.float32),
    mesh=_mesh,
    scratch_types=[
        pltpu.VMEM((VOCAB * WPR,), jnp.int32),
        pltpu.VMEM((VCHUNK, DIM), jnp.float32),
        pltpu.VMEM((VCHUNK * WPR,), jnp.int32),
        pltpu.VMEM((SENT, BPW), jnp.int32),
        pltpu.VMEM((BPW * (SENT + 1) + _L,), jnp.int32),
        pltpu.VMEM((BPW, DIM), jnp.float32),
        pltpu.VMEM_SHARED((VOCAB * WPR,), jnp.int32),
        pltpu.SemaphoreType.DMA,
    ],
    compiler_params=pltpu.CompilerParams(needs_layout_passes=False),
)
def _sc_pool(emb_hbm, xt_hbm, out_hbm, table_v, stage_v, slice_v,
             idxt_v, idx_v, pool_v, spk_v, sem):
    s = lax.axis_index("s")
    w = s * _NC + lax.axis_index("c")
    idx_cp = pltpu.async_copy(xt_hbm.at[:, pl.ds(w * BPW, BPW)], idxt_v, sem)

    start = jnp.minimum(s * VCHUNK, VOCAB - VCHUNK)
    pltpu.sync_copy(emb_hbm.at[pl.ds(start, VCHUNK)], stage_v)

    def pack_row(r, carry):
        vs = [stage_v[r, pl.ds(k * _L, _L)] for k in range(4)]
        for k in range(2):
            pk = plsc.bitcast(
                plsc.pack(vs[2 * k], vs[2 * k + 1],
                          format=plsc.PackFormat.INTERLEAVED),
                jnp.int32)
            slice_v[pl.ds(r * WPR + k * _L, _L)] = pk
        return carry

    lax.fori_loop(0, VCHUNK, pack_row, jnp.int32(0))
    pltpu.sync_copy(slice_v, spk_v.at[pl.ds(start * WPR, VCHUNK * WPR)])

    idx_cp.wait()
    srow = SENT + 1
    giota = [(lax.iota(jnp.int32, _L) + g * _L) * srow for g in range(NBG)]

    def tr_body(t, carry):
        for g in range(NBG):
            v = idxt_v[t, pl.ds(g * _L, _L)]
            plsc.store_scatter(idx_v, [giota[g] + t], v)
        return carry

    lax.fori_loop(0, SENT, tr_body, jnp.int32(0))

    plsc.subcore_barrier()
    pltpu.sync_copy(spk_v, table_v)

    def accum(b):
        accs = [jnp.zeros((_L,), jnp.float32) for _ in range(4)]
        for off, js in _TGROUPS:
            toks = idx_v[pl.ds(b * srow + off, _L)]
            for j in js:
                base = toks[j] * WPR
                for k in range(2):
                    v = table_v[pl.ds(base + k * _L, _L)]
                    lo = lax.bitcast_convert_type(v << 16, jnp.float32)
                    hi = lax.bitcast_convert_type(v, jnp.float32)
                    accs[2 * k] = accs[2 * k] + lo
                    accs[2 * k + 1] = accs[2 * k + 1] + hi
        return accs

    def body(i, carry):
        for u in range(2):
            b = i * 2 + u
            accs = accum(b)
            for k in range(4):
                pool_v[b, pl.ds(k * _L, _L)] = accs[k]
        return carry

    lax.fori_loop(0, BPW // 2, body, jnp.int32(0))
    pltpu.sync_copy(pool_v, out_hbm.at[pl.ds(w * BPW, BPW)])


def _head_body(p_ref, w_ref, b_ref, o_ref):
    logits = lax.dot_general(
        w_ref[...], p_ref[...], (((1,), (1,)), ((), ())),
        preferred_element_type=jnp.float32)
    logits = logits * (1.0 / SENT) + b_ref[...]
    m = jnp.max(logits, axis=0, keepdims=True)
    e = jnp.exp(logits - m)
    o_ref[...] = e * (1.0 / jnp.sum(e, axis=0, keepdims=True))


_head = pl.pallas_call(
    _head_body,
    out_shape=jax.ShapeDtypeStruct((OUT, B), jnp.float32),
)


def kernel(x, emb, W, b):
    pooled_sum = _sc_pool(emb.astype(jnp.float32), x.astype(jnp.int32).T)
    return _head(pooled_sum, W.astype(jnp.float32), b.reshape(OUT, 1)).T


# speedup vs baseline: 1.0348x; 1.0348x over previous
import functools

import jax
import jax.numpy as jnp
from jax import lax
from jax.experimental import pallas as pl
from jax.experimental.pallas import tpu as pltpu
from jax.experimental.pallas import tpu_sc as plsc

VOCAB = 1000
DIM = 64
SENT = 50
B = 4096
OUT = SENT + 1
WPR = DIM // 2
VCHUNK = 64

try:
    _info = plsc.get_sparse_core_info()
    _NC, _NS, _L = _info.num_cores, _info.num_subcores, _info.num_lanes
except Exception:
    _NC, _NS, _L = 2, 16, 16

NW = _NC * _NS
BPW = B // NW
NBG = BPW // _L

_mesh = plsc.VectorSubcoreMesh(
    core_axis_name="c", subcore_axis_name="s",
    num_cores=_NC, num_subcores=_NS,
)

_TGROUPS = [(0, range(_L)), (_L, range(_L)), (2 * _L, range(_L)),
            (3 * _L, range(SENT - 3 * _L))]


@functools.partial(
    pl.kernel,
    out_type=jax.ShapeDtypeStruct((B, DIM), jnp.float32),
    mesh=_mesh,
    scratch_types=[
        pltpu.VMEM((VOCAB * WPR,), jnp.int32),
        pltpu.VMEM((VCHUNK, DIM), jnp.float32),
        pltpu.VMEM((VCHUNK * WPR,), jnp.int32),
        pltpu.VMEM((SENT, BPW), jnp.int32),
        pltpu.VMEM((BPW * (SENT + 1) + _L,), jnp.int32),
        pltpu.VMEM((BPW, DIM), jnp.float32),
        pltpu.VMEM_SHARED((VOCAB * WPR,), jnp.int32),
        pltpu.SemaphoreType.DMA,
    ],
    compiler_params=pltpu.CompilerParams(needs_layout_passes=False),
)
def _sc_pool(emb_hbm, xt_hbm, out_hbm, table_v, stage_v, slice_v,
             idxt_v, idx_v, pool_v, spk_v, sem):
    s = lax.axis_index("s")
    w = s * _NC + lax.axis_index("c")
    idx_cp = pltpu.async_copy(xt_hbm.at[:, pl.ds(w * BPW, BPW)], idxt_v, sem)

    start = jnp.minimum(s * VCHUNK, VOCAB - VCHUNK)
    pltpu.sync_copy(emb_hbm.at[pl.ds(start, VCHUNK)], stage_v)

    def pack_row(r, carry):
        vs = [stage_v[r, pl.ds(k * _L, _L)] for k in range(4)]
        for k in range(2):
            pk = plsc.bitcast(
                plsc.pack(vs[2 * k], vs[2 * k + 1],
                          format=plsc.PackFormat.INTERLEAVED),
                jnp.int32)
            slice_v[pl.ds(r * WPR + k * _L, _L)] = pk
        return carry

    lax.fori_loop(0, VCHUNK, pack_row, jnp.int32(0))
    pltpu.sync_copy(slice_v, spk_v.at[pl.ds(start * WPR, VCHUNK * WPR)])

    plsc.subcore_barrier()
    table_cp = pltpu.async_copy(spk_v, table_v, sem)

    idx_cp.wait()
    srow = SENT + 1
    giota = [(lax.iota(jnp.int32, _L) + g * _L) * srow for g in range(NBG)]

    def tr_body(t, carry):
        for g in range(NBG):
            v = idxt_v[t, pl.ds(g * _L, _L)]
            plsc.store_scatter(idx_v, [giota[g] + t], v)
        return carry

    lax.fori_loop(0, SENT, tr_body, jnp.int32(0))
    table_cp.wait()

    def accum(b):
        accs = [jnp.zeros((_L,), jnp.float32) for _ in range(4)]
        for off, js in _TGROUPS:
            toks = idx_v[pl.ds(b * srow + off, _L)]
            for j in js:
                base = toks[j] * WPR
                for k in range(2):
                    v = table_v[pl.ds(base + k * _L, _L)]
                    lo = lax.bitcast_convert_type(v << 16, jnp.float32)
                    hi = lax.bitcast_convert_type(v, jnp.float32)
                    accs[2 * k] = accs[2 * k] + lo
                    accs[2 * k + 1] = accs[2 * k + 1] + hi
        return accs

    def body(i, carry):
        for u in range(2):
            b = i * 2 + u
            accs = accum(b)
            for k in range(4):
                pool_v[b, pl.ds(k * _L, _L)] = accs[k]
        return carry

    lax.fori_loop(0, BPW // 2, body, jnp.int32(0))
    pltpu.sync_copy(pool_v, out_hbm.at[pl.ds(w * BPW, BPW)])


def _head_body(p_ref, w_ref, b_ref, o_ref):
    logits = lax.dot_general(
        w_ref[...], p_ref[...], (((1,), (1,)), ((), ())),
        preferred_element_type=jnp.float32)
    logits = logits * (1.0 / SENT) + b_ref[...]
    m = jnp.max(logits, axis=0, keepdims=True)
    e = jnp.exp(logits - m)
    o_ref[...] = e * (1.0 / jnp.sum(e, axis=0, keepdims=True))


_head = pl.pallas_call(
    _head_body,
    out_shape=jax.ShapeDtypeStruct((OUT, B), jnp.float32),
)


def kernel(x, emb, W, b):
    pooled_sum = _sc_pool(emb.astype(jnp.float32), x.astype(jnp.int32).T)
    return _head(pooled_sum, W.astype(jnp.float32), b.reshape(OUT, 1)).T


# speedup vs baseline: 1.0382x; 1.0032x over previous
import functools

import jax
import jax.numpy as jnp
from jax import lax
from jax.experimental import pallas as pl
from jax.experimental.pallas import tpu as pltpu
from jax.experimental.pallas import tpu_sc as plsc

VOCAB = 1000
DIM = 64
SENT = 50
B = 4096
OUT = SENT + 1
WPR = DIM // 2
VCHUNK = 64

try:
    _info = plsc.get_sparse_core_info()
    _NC, _NS, _L = _info.num_cores, _info.num_subcores, _info.num_lanes
except Exception:
    _NC, _NS, _L = 2, 16, 16

NW = _NC * _NS
BPW = B // NW
NBG = BPW // _L

_mesh = plsc.VectorSubcoreMesh(
    core_axis_name="c", subcore_axis_name="s",
    num_cores=_NC, num_subcores=_NS,
)

_TGROUPS = [(0, range(_L)), (_L, range(_L)), (2 * _L, range(_L)),
            (3 * _L, range(SENT - 3 * _L))]


@functools.partial(
    pl.kernel,
    out_type=jax.ShapeDtypeStruct((B, DIM), jnp.float32),
    mesh=_mesh,
    scratch_types=[
        pltpu.VMEM((VOCAB * WPR,), jnp.int32),
        pltpu.VMEM((VCHUNK, DIM), jnp.float32),
        pltpu.VMEM((VCHUNK * WPR,), jnp.int32),
        pltpu.VMEM((SENT, BPW), jnp.int32),
        pltpu.VMEM((BPW * (SENT + 1) + _L,), jnp.int32),
        pltpu.VMEM((BPW, DIM), jnp.float32),
        pltpu.VMEM_SHARED((VOCAB * WPR,), jnp.int32),
        pltpu.SemaphoreType.DMA,
    ],
    compiler_params=pltpu.CompilerParams(needs_layout_passes=False),
)
def _sc_pool(emb_hbm, xt_hbm, out_hbm, table_v, stage_v, slice_v,
             idxt_v, idx_v, pool_v, spk_v, sem):
    s = lax.axis_index("s")
    w = s * _NC + lax.axis_index("c")
    idx_cp = pltpu.async_copy(xt_hbm.at[:, pl.ds(w * BPW, BPW)], idxt_v, sem)

    start = jnp.minimum(s * VCHUNK, VOCAB - VCHUNK)
    pltpu.sync_copy(emb_hbm.at[pl.ds(start, VCHUNK)], stage_v)

    def pack_row(r, carry):
        vs = [stage_v[r, pl.ds(k * _L, _L)] for k in range(4)]
        for k in range(2):
            pk = plsc.bitcast(
                plsc.pack(vs[2 * k], vs[2 * k + 1],
                          format=plsc.PackFormat.INTERLEAVED),
                jnp.int32)
            slice_v[pl.ds(r * WPR + k * _L, _L)] = pk
        return carry

    lax.fori_loop(0, VCHUNK, pack_row, jnp.int32(0))
    pltpu.sync_copy(slice_v, spk_v.at[pl.ds(start * WPR, VCHUNK * WPR)])

    plsc.subcore_barrier()
    table_cp = pltpu.async_copy(spk_v, table_v, sem)

    idx_cp.wait()
    srow = SENT + 1
    giota = [(lax.iota(jnp.int32, _L) + g * _L) * srow for g in range(NBG)]

    def tr_body(t, carry):
        for g in range(NBG):
            v = idxt_v[t, pl.ds(g * _L, _L)]
            plsc.store_scatter(idx_v, [giota[g] + t], v)
        return carry

    lax.fori_loop(0, SENT, tr_body, jnp.int32(0))
    table_cp.wait()

    def accum(b):
        accs = [jnp.zeros((_L,), jnp.float32) for _ in range(4)]
        for off, js in _TGROUPS:
            toks = idx_v[pl.ds(b * srow + off, _L)]
            for j in js:
                base = toks[j] * WPR
                for k in range(2):
                    v = table_v[pl.ds(base + k * _L, _L)]
                    lo = lax.bitcast_convert_type(v << 16, jnp.float32)
                    hi = lax.bitcast_convert_type(v, jnp.float32)
                    accs[2 * k] = accs[2 * k] + lo
                    accs[2 * k + 1] = accs[2 * k + 1] + hi
        return accs

    def body(b, carry):
        accs = accum(b)
        for k in range(4):
            pool_v[b, pl.ds(k * _L, _L)] = accs[k]
        return carry

    lax.fori_loop(0, BPW, body, jnp.int32(0))
    pltpu.sync_copy(pool_v, out_hbm.at[pl.ds(w * BPW, BPW)])


def _head_body(p_ref, w_ref, b_ref, o_ref):
    logits = lax.dot_general(
        w_ref[...], p_ref[...], (((1,), (1,)), ((), ())),
        preferred_element_type=jnp.float32)
    logits = logits * (1.0 / SENT) + b_ref[...]
    m = jnp.max(logits, axis=0, keepdims=True)
    e = jnp.exp(logits - m)
    o_ref[...] = e * (1.0 / jnp.sum(e, axis=0, keepdims=True))


_head = pl.pallas_call(
    _head_body,
    out_shape=jax.ShapeDtypeStruct((OUT, B), jnp.float32),
)


def kernel(x, emb, W, b):
    pooled_sum = _sc_pool(emb.astype(jnp.float32), x.astype(jnp.int32).T)
    return _head(pooled_sum, W.astype(jnp.float32), b.reshape(OUT, 1)).T


# speedup vs baseline: 1.0459x; 1.0075x over previous
import functools

import jax
import jax.numpy as jnp
from jax import lax
from jax.experimental import pallas as pl
from jax.experimental.pallas import tpu as pltpu
from jax.experimental.pallas import tpu_sc as plsc

VOCAB = 1000
DIM = 64
SENT = 50
B = 4096
OUT = SENT + 1
WPR = DIM // 2
VCHUNK = 64

try:
    _info = plsc.get_sparse_core_info()
    _NC, _NS, _L = _info.num_cores, _info.num_subcores, _info.num_lanes
except Exception:
    _NC, _NS, _L = 2, 16, 16

NW = _NC * _NS
BPW = B // NW
NBG = BPW // _L

_mesh = plsc.VectorSubcoreMesh(
    core_axis_name="c", subcore_axis_name="s",
    num_cores=_NC, num_subcores=_NS,
)

_TGROUPS = [(0, range(_L)), (_L, range(_L)), (2 * _L, range(_L)),
            (3 * _L, range(SENT - 3 * _L))]


@functools.partial(
    pl.kernel,
    out_type=jax.ShapeDtypeStruct((B, DIM), jnp.float32),
    mesh=_mesh,
    scratch_types=[
        pltpu.VMEM((VOCAB * WPR,), jnp.int32),
        pltpu.VMEM((VCHUNK, DIM), jnp.float32),
        pltpu.VMEM((VCHUNK * WPR,), jnp.int32),
        pltpu.VMEM((SENT, BPW), jnp.int32),
        pltpu.VMEM((BPW * (SENT + 1) + _L,), jnp.int32),
        pltpu.VMEM((BPW, DIM), jnp.float32),
        pltpu.VMEM_SHARED((VOCAB * WPR,), jnp.int32),
        pltpu.SemaphoreType.DMA,
    ],
    compiler_params=pltpu.CompilerParams(needs_layout_passes=False),
)
def _sc_pool(emb_hbm, xt_hbm, out_hbm, table_v, stage_v, slice_v,
             idxt_v, idx_v, pool_v, spk_v, sem):
    s = lax.axis_index("s")
    w = s * _NC + lax.axis_index("c")
    idx_cp = pltpu.async_copy(xt_hbm.at[:, pl.ds(w * BPW, BPW)], idxt_v, sem)

    start = jnp.minimum(s * VCHUNK, VOCAB - VCHUNK)
    pltpu.sync_copy(emb_hbm.at[pl.ds(start, VCHUNK)], stage_v)

    def pack_row(r, carry):
        vs = [stage_v[r, pl.ds(k * _L, _L)] for k in range(4)]
        for k in range(2):
            pk = plsc.bitcast(
                plsc.pack(vs[2 * k], vs[2 * k + 1],
                          format=plsc.PackFormat.INTERLEAVED),
                jnp.int32)
            slice_v[pl.ds(r * WPR + k * _L, _L)] = pk
        return carry

    lax.fori_loop(0, VCHUNK, pack_row, jnp.int32(0))
    pltpu.sync_copy(slice_v, spk_v.at[pl.ds(start * WPR, VCHUNK * WPR)])

    plsc.subcore_barrier()
    table_cp = pltpu.async_copy(spk_v, table_v, sem)

    idx_cp.wait()
    srow = SENT + 1
    giota = [(lax.iota(jnp.int32, _L) + g * _L) * srow for g in range(NBG)]

    def tr_body(t, carry):
        for g in range(NBG):
            v = idxt_v[t, pl.ds(g * _L, _L)]
            plsc.store_scatter(idx_v, [giota[g] + t], v)
        return carry

    lax.fori_loop(0, SENT, tr_body, jnp.int32(0))
    table_cp.wait()

    def accum(b):
        accs = [jnp.zeros((_L,), jnp.float32) for _ in range(4)]
        for off, js in _TGROUPS:
            toks = idx_v[pl.ds(b * srow + off, _L)]
            for j in js:
                base = toks[j] * WPR
                for k in range(2):
                    v = table_v[pl.ds(base + k * _L, _L)]
                    lo = lax.bitcast_convert_type(v << 16, jnp.float32)
                    hi = lax.bitcast_convert_type(v, jnp.float32)
                    accs[2 * k] = accs[2 * k] + lo
                    accs[2 * k + 1] = accs[2 * k + 1] + hi
        return accs

    def body(b, carry):
        accs = accum(b)
        for k in range(4):
            pool_v[b, pl.ds(k * _L, _L)] = accs[k]
        return carry

    lax.fori_loop(0, BPW, body, jnp.int32(0))
    pltpu.sync_copy(pool_v, out_hbm.at[pl.ds(w * BPW, BPW)])


def _head_body(p_ref, w_ref, b_ref, o_ref):
    logits = lax.dot_general(
        w_ref[...], p_ref[...], (((1,), (1,)), ((), ())),
        preferred_element_type=jnp.float32)
    logits = logits * (1.0 / SENT) + b_ref[...]
    m = jnp.max(logits, axis=0, keepdims=True)
    e = jnp.exp(logits - m)
    o_ref[...] = e * (1.0 / jnp.sum(e, axis=0, keepdims=True))


_HB = B // 2
_head = pl.pallas_call(
    _head_body,
    grid=(B // _HB,),
    in_specs=[
        pl.BlockSpec((_HB, DIM), lambda i: (i, 0)),
        pl.BlockSpec((OUT, DIM), lambda i: (0, 0)),
        pl.BlockSpec((OUT, 1), lambda i: (0, 0)),
    ],
    out_specs=pl.BlockSpec((OUT, _HB), lambda i: (0, i)),
    out_shape=jax.ShapeDtypeStruct((OUT, B), jnp.float32),
)


def kernel(x, emb, W, b):
    pooled_sum = _sc_pool(emb.astype(jnp.float32), x.astype(jnp.int32).T)
    return _head(pooled_sum, W.astype(jnp.float32), b.reshape(OUT, 1)).T
